# Initial kernel scaffold; baseline (speedup 1.0000x reference)
import functools

import jax
import jax.numpy as jnp
from jax import lax
from jax.experimental import pallas as pl
from jax.experimental.pallas import tpu as pltpu
from jax.experimental.pallas import tpu_sc as plsc

L = 16


def _build(B, R, N_PAD, CAP_PAD, b_per_w):
    mesh = plsc.VectorSubcoreMesh(core_axis_name="c", subcore_axis_name="s")

    @functools.partial(
        pl.kernel,
        mesh=mesh,
        out_type=jax.ShapeDtypeStruct((B,), jnp.float32),
        scratch_types=[
            pltpu.VMEM((b_per_w, 2), jnp.int32),
            pltpu.VMEM((R, N_PAD), jnp.float32),
            pltpu.VMEM((R, N_PAD), jnp.float32),
            pltpu.VMEM((CAP_PAD,), jnp.int32),
            pltpu.VMEM((N_PAD,), jnp.int32),
            pltpu.VMEM((b_per_w,), jnp.float32),
        ],
    )
    def demand_kernel(batch_hbm, as_hbm, bs_hbm, cap_hbm, out_hbm,
                      batch_v, as_v, bs_v, cap_v, mask_v, out_v):
        wid = lax.axis_index("s") * 2 + lax.axis_index("c")
        base = wid * b_per_w

        pltpu.sync_copy(batch_hbm.at[pl.ds(base, b_per_w)], batch_v)
        pltpu.sync_copy(as_hbm, as_v)
        pltpu.sync_copy(bs_hbm, bs_v)
        pltpu.sync_copy(cap_hbm, cap_v)

        zeros = jnp.zeros((L,), jnp.int32)
        ones = jnp.ones((L,), jnp.int32)

        def zero_body(k, carry):
            mask_v[pl.ds(k * L, L)] = zeros
            return carry
        lax.fori_loop(0, N_PAD // L, zero_body, 0, unroll=4)

        def scat_body(k, carry):
            idx = cap_v[pl.ds(k * L, L)]
            plsc.store_scatter(mask_v, [idx], ones)
            return carry
        lax.fori_loop(0, CAP_PAD // L, scat_body, 0)

        lane = jax.lax.iota(jnp.int32, L)

        def body(k, carry):
            rows = lane + k * L
            iv = plsc.load_gather(batch_v, [rows, zeros])
            jv = plsc.load_gather(batch_v, [rows, ones])
            mi = plsc.load_gather(mask_v, [iv])
            mj = plsc.load_gather(mask_v, [jv])
            r = jnp.bitwise_or(mi, mj)
            ai = plsc.load_gather(as_v, [r, iv])
            aj = plsc.load_gather(as_v, [r, jv])
            bi = plsc.load_gather(bs_v, [r, iv])
            bj = plsc.load_gather(bs_v, [r, jv])
            out_v[pl.ds(k * L, L)] = ai * aj + bi + bj
            return carry
        lax.fori_loop(0, b_per_w // L, body, 0, unroll=2)

        pltpu.sync_copy(out_v, out_hbm.at[pl.ds(base, b_per_w)])

    return demand_kernel


def kernel(batch, As, Bs, capital_ids):
    B = batch.shape[0]
    R, N = As.shape
    N_PAD = ((N + L - 1) // L) * L
    CAP = capital_ids.shape[0]
    CAP_PAD = ((CAP + L - 1) // L) * L
    NW = 32
    b_per_w = B // NW

    batch32 = batch.astype(jnp.int32)
    cap32 = capital_ids.astype(jnp.int32)
    cap_pad = jnp.concatenate(
        [cap32, jnp.broadcast_to(cap32[0], (CAP_PAD - CAP,))])
    as_pad = jnp.pad(As, ((0, 0), (0, N_PAD - N)))
    bs_pad = jnp.pad(Bs, ((0, 0), (0, N_PAD - N)))

    fn = _build(B, R, N_PAD, CAP_PAD, b_per_w)
    return fn(batch32, as_pad, bs_pad, cap_pad)


# speedup vs baseline: 7.8617x; 7.8617x over previous
import functools

import jax
import jax.numpy as jnp
from jax import lax
from jax.experimental import pallas as pl
from jax.experimental.pallas import tpu as pltpu
from jax.experimental.pallas import tpu_sc as plsc

L = 16


def _build(B, R, N_PAD, CAP_PAD, b_per_w):
    mesh = plsc.VectorSubcoreMesh(core_axis_name="c", subcore_axis_name="s")

    @functools.partial(
        pl.kernel,
        mesh=mesh,
        out_type=jax.ShapeDtypeStruct((B,), jnp.float32),
        compiler_params=pltpu.CompilerParams(needs_layout_passes=False),
        scratch_types=[
            pltpu.VMEM((b_per_w, 2), jnp.int32),
            pltpu.VMEM((R, N_PAD), jnp.float32),
            pltpu.VMEM((R, N_PAD), jnp.float32),
            pltpu.VMEM((CAP_PAD,), jnp.int32),
            pltpu.VMEM((N_PAD,), jnp.int32),
            pltpu.VMEM((b_per_w,), jnp.float32),
        ],
    )
    def demand_kernel(batch_hbm, as_hbm, bs_hbm, cap_hbm, out_hbm,
                      batch_v, as_v, bs_v, cap_v, mask_v, out_v):
        wid = lax.axis_index("s") * 2 + lax.axis_index("c")
        base = wid * b_per_w

        pltpu.sync_copy(batch_hbm.at[pl.ds(base, b_per_w)], batch_v)
        pltpu.sync_copy(as_hbm, as_v)
        pltpu.sync_copy(bs_hbm, bs_v)
        pltpu.sync_copy(cap_hbm, cap_v)

        zeros = jnp.zeros((L,), jnp.int32)
        ones = jnp.ones((L,), jnp.int32)

        def zero_body(k, carry):
            mask_v[pl.ds(k * L, L)] = zeros
            return carry
        lax.fori_loop(0, N_PAD // L, zero_body, 0, unroll=4)

        def scat_body(k, carry):
            idx = cap_v[pl.ds(k * L, L)]
            plsc.store_scatter(mask_v, [idx], ones)
            return carry
        lax.fori_loop(0, CAP_PAD // L, scat_body, 0)

        lane = jax.lax.iota(jnp.int32, L)

        def body(k, carry):
            rows = lane + k * L
            iv = plsc.load_gather(batch_v, [rows, zeros])
            jv = plsc.load_gather(batch_v, [rows, ones])
            mi = plsc.load_gather(mask_v, [iv])
            mj = plsc.load_gather(mask_v, [jv])
            r = jnp.bitwise_or(mi, mj)
            ai = plsc.load_gather(as_v, [r, iv])
            aj = plsc.load_gather(as_v, [r, jv])
            bi = plsc.load_gather(bs_v, [r, iv])
            bj = plsc.load_gather(bs_v, [r, jv])
            out_v[pl.ds(k * L, L)] = ai * aj + bi + bj
            return carry
        lax.fori_loop(0, b_per_w // L, body, 0, unroll=2)

        pltpu.sync_copy(out_v, out_hbm.at[pl.ds(base, b_per_w)])

    return demand_kernel


def kernel(batch, As, Bs, capital_ids):
    B = batch.shape[0]
    R, N = As.shape
    N_PAD = ((N + L - 1) // L) * L
    CAP = capital_ids.shape[0]
    CAP_PAD = ((CAP + L - 1) // L) * L
    NW = 32
    b_per_w = B // NW

    batch32 = batch.astype(jnp.int32)
    cap32 = capital_ids.astype(jnp.int32)
    cap_pad = jnp.concatenate(
        [cap32, jnp.broadcast_to(cap32[0], (CAP_PAD - CAP,))])
    as_pad = jnp.pad(As, ((0, 0), (0, N_PAD - N)))
    bs_pad = jnp.pad(Bs, ((0, 0), (0, N_PAD - N)))

    fn = _build(B, R, N_PAD, CAP_PAD, b_per_w)
    return fn(batch32, as_pad, bs_pad, cap_pad)


# speedup vs baseline: 8.3764x; 1.0655x over previous
import functools

import jax
import jax.numpy as jnp
from jax import lax
from jax.experimental import pallas as pl
from jax.experimental.pallas import tpu as pltpu
from jax.experimental.pallas import tpu_sc as plsc

L = 16


def _build(B, R, N_PAD, CAP_PAD, b_per_w):
    mesh = plsc.VectorSubcoreMesh(core_axis_name="c", subcore_axis_name="s")

    @functools.partial(
        pl.kernel,
        mesh=mesh,
        out_type=jax.ShapeDtypeStruct((B,), jnp.float32),
        compiler_params=pltpu.CompilerParams(needs_layout_passes=False),
        scratch_types=[
            pltpu.VMEM((b_per_w, 2), jnp.int32),
            pltpu.VMEM((R, N_PAD), jnp.float32),
            pltpu.VMEM((R, N_PAD), jnp.float32),
            pltpu.VMEM((CAP_PAD,), jnp.int32),
            pltpu.VMEM((N_PAD,), jnp.int32),
            pltpu.VMEM((b_per_w,), jnp.float32),
            pltpu.SemaphoreType.DMA,
        ],
    )
    def demand_kernel(batch_hbm, as_hbm, bs_hbm, cap_hbm, out_hbm,
                      batch_v, as_v, bs_v, cap_v, mask_v, out_v, sem):
        wid = lax.axis_index("s") * 2 + lax.axis_index("c")
        base = wid * b_per_w

        c0 = pltpu.async_copy(batch_hbm.at[pl.ds(base, b_per_w)], batch_v, sem)
        c1 = pltpu.async_copy(as_hbm, as_v, sem)
        c2 = pltpu.async_copy(bs_hbm, bs_v, sem)
        c3 = pltpu.async_copy(cap_hbm, cap_v, sem)

        zeros = jnp.zeros((L,), jnp.int32)
        ones = jnp.ones((L,), jnp.int32)

        def zero_body(k, carry):
            mask_v[pl.ds(k * L, L)] = zeros
            return carry
        lax.fori_loop(0, N_PAD // L, zero_body, 0, unroll=8)

        c0.wait()
        c1.wait()
        c2.wait()
        c3.wait()

        for k in range(CAP_PAD // L):
            idx = cap_v[pl.ds(k * L, L)]
            plsc.store_scatter(mask_v, [idx], ones)

        lane = jax.lax.iota(jnp.int32, L)

        def body(k, carry):
            rows = lane + k * L
            iv = plsc.load_gather(batch_v, [rows, zeros])
            jv = plsc.load_gather(batch_v, [rows, ones])
            mi = plsc.load_gather(mask_v, [iv])
            mj = plsc.load_gather(mask_v, [jv])
            r = jnp.bitwise_or(mi, mj)
            ai = plsc.load_gather(as_v, [r, iv])
            aj = plsc.load_gather(as_v, [r, jv])
            bi = plsc.load_gather(bs_v, [r, iv])
            bj = plsc.load_gather(bs_v, [r, jv])
            out_v[pl.ds(k * L, L)] = ai * aj + bi + bj
            return carry
        lax.fori_loop(0, b_per_w // L, body, 0, unroll=8)

        pltpu.sync_copy(out_v, out_hbm.at[pl.ds(base, b_per_w)])

    return demand_kernel


def kernel(batch, As, Bs, capital_ids):
    B = batch.shape[0]
    R, N = As.shape
    N_PAD = ((N + L - 1) // L) * L
    CAP = capital_ids.shape[0]
    CAP_PAD = ((CAP + L - 1) // L) * L
    NW = 32
    b_per_w = B // NW

    batch32 = batch.astype(jnp.int32)
    cap32 = capital_ids.astype(jnp.int32)
    cap_pad = jnp.concatenate(
        [cap32, jnp.broadcast_to(cap32[0], (CAP_PAD - CAP,))])
    as_pad = jnp.pad(As, ((0, 0), (0, N_PAD - N)))
    bs_pad = jnp.pad(Bs, ((0, 0), (0, N_PAD - N)))

    fn = _build(B, R, N_PAD, CAP_PAD, b_per_w)
    return fn(batch32, as_pad, bs_pad, cap_pad)


# speedup vs baseline: 9.4060x; 1.1229x over previous
import functools

import jax
import jax.numpy as jnp
from jax import lax
from jax.experimental import pallas as pl
from jax.experimental.pallas import tpu as pltpu
from jax.experimental.pallas import tpu_sc as plsc

L = 16


def _build(B, R, N, CAP, b_per_w):
    N_PAD = ((N + L - 1) // L) * L
    CAP_PAD = ((CAP + L - 1) // L) * L
    mesh = plsc.VectorSubcoreMesh(core_axis_name="c", subcore_axis_name="s")

    @functools.partial(
        pl.kernel,
        mesh=mesh,
        out_type=jax.ShapeDtypeStruct((B,), jnp.float32),
        compiler_params=pltpu.CompilerParams(needs_layout_passes=False),
        scratch_types=[
            pltpu.VMEM((b_per_w, 2), jnp.int32),
            pltpu.VMEM((R, N), jnp.float32),
            pltpu.VMEM((R, N), jnp.float32),
            pltpu.VMEM((CAP_PAD,), jnp.int32),
            pltpu.VMEM((N_PAD,), jnp.int32),
            pltpu.VMEM((b_per_w,), jnp.float32),
            pltpu.SemaphoreType.DMA,
        ],
    )
    def demand_kernel(batch_hbm, as_hbm, bs_hbm, cap_hbm, out_hbm,
                      batch_v, as_v, bs_v, cap_v, mask_v, out_v, sem):
        wid = lax.axis_index("s") * 2 + lax.axis_index("c")
        base = wid * b_per_w

        c0 = pltpu.async_copy(batch_hbm.at[pl.ds(base, b_per_w)], batch_v, sem)
        c1 = pltpu.async_copy(as_hbm, as_v, sem)
        c2 = pltpu.async_copy(bs_hbm, bs_v, sem)
        c3 = pltpu.async_copy(cap_hbm, cap_v.at[pl.ds(0, CAP)], sem)

        zeros = jnp.zeros((L,), jnp.int32)
        ones = jnp.ones((L,), jnp.int32)
        lane = jax.lax.iota(jnp.int32, L)

        def zero_body(k, carry):
            mask_v[pl.ds(k * L, L)] = zeros
            return carry
        lax.fori_loop(0, N_PAD // L, zero_body, 0, unroll=8)

        c0.wait()
        c1.wait()
        c2.wait()
        c3.wait()

        for k in range(CAP_PAD // L):
            idx = cap_v[pl.ds(k * L, L)]
            if (k + 1) * L <= CAP:
                plsc.store_scatter(mask_v, [idx], ones)
            else:
                tail = jnp.full((L,), CAP - k * L, jnp.int32)
                plsc.store_scatter(mask_v, [idx], ones, mask=lane < tail)

        def body(k, carry):
            rows = lane + k * L
            iv = plsc.load_gather(batch_v, [rows, zeros])
            jv = plsc.load_gather(batch_v, [rows, ones])
            mi = plsc.load_gather(mask_v, [iv])
            mj = plsc.load_gather(mask_v, [jv])
            r = jnp.bitwise_or(mi, mj)
            ai = plsc.load_gather(as_v, [r, iv])
            aj = plsc.load_gather(as_v, [r, jv])
            bi = plsc.load_gather(bs_v, [r, iv])
            bj = plsc.load_gather(bs_v, [r, jv])
            out_v[pl.ds(k * L, L)] = ai * aj + bi + bj
            return carry
        lax.fori_loop(0, b_per_w // L, body, 0, unroll=8)

        pltpu.sync_copy(out_v, out_hbm.at[pl.ds(base, b_per_w)])

    return demand_kernel


def kernel(batch, As, Bs, capital_ids):
    B = batch.shape[0]
    R, N = As.shape
    CAP = capital_ids.shape[0]
    NW = 32
    b_per_w = B // NW

    fn = _build(B, R, N, CAP, b_per_w)
    return fn(batch.astype(jnp.int32), As, Bs, capital_ids.astype(jnp.int32))
